# baseline (speedup 1.0000x reference)
import functools

import jax
import jax.numpy as jnp
from jax import lax
from jax.experimental import pallas as pl
from jax.experimental.pallas import tpu as pltpu
from jax.experimental.pallas import tpu_sc as plsc

B = 16384
F = 27
IDX = 26
VOCAB = 1000000
EMB = 16
NPAIR = EMB // 2
HID = 128
OUT = 64

BLK = 8192
CH = 65536
SLAB = 16 * CH
NCOL = (VOCAB + CH - 1) // CH


def _rne16(v):
    return (v + jnp.uint32(0x7FFF) + ((v >> 16) & jnp.uint32(1))) & jnp.uint32(
        0xFFFF0000)


def _relayout_body(t_ref, xt_ref, o_ref, idx_ref, pk_ref, sem):
    c = pl.program_id(0)
    cur = c % 2
    @pl.when(c == 0)
    def _():
        idx_ref[...] = xt_ref[IDX, :].astype(jnp.int32)
    u = pltpu.bitcast(t_ref[...], jnp.uint32)
    lo = u[:NPAIR, :]
    hi = u[NPAIR:, :]
    packed = pltpu.bitcast(_rne16(hi) | (_rne16(lo) >> 16), jnp.float32)
    @pl.when(c >= 2)
    def _():
        for t in range(NPAIR):
            pltpu.make_async_copy(
                pk_ref.at[cur, t], o_ref.at[pl.ds(t * CH, CH)], sem).wait()
    pk_ref[pl.ds(cur, 1)] = packed.reshape(1, NPAIR, CH)
    for t in range(NPAIR):
        pltpu.make_async_copy(
            pk_ref.at[cur, t],
            o_ref.at[pl.ds(t * SLAB + c * CH, CH)], sem).start()
    @pl.when(c == NCOL - 1)
    def _():
        for t in range(2 * NPAIR):
            pltpu.make_async_copy(
                pk_ref.at[0, t % NPAIR], o_ref.at[pl.ds((t % NPAIR) * CH, CH)],
                sem).wait()


def _relayout(tablet, xt):
    return pl.pallas_call(
        _relayout_body,
        grid=(NCOL,),
        in_specs=[
            pl.BlockSpec((EMB, CH), lambda c: (0, c)),
            pl.BlockSpec((F, B), lambda c: (0, 0)),
        ],
        out_specs=[
            pl.BlockSpec(memory_space=pl.ANY),
            pl.BlockSpec((B,), lambda c: (0,)),
        ],
        out_shape=[
            jax.ShapeDtypeStruct((NPAIR * SLAB,), jnp.float32),
            jax.ShapeDtypeStruct((B,), jnp.int32),
        ],
        scratch_shapes=[
            pltpu.VMEM((2, NPAIR, CH), jnp.float32),
            pltpu.SemaphoreType.DMA,
        ],
    )(tablet, xt)


def _build_gather():
    info = plsc.get_sparse_core_info()
    nc, ns, nl = info.num_cores, info.num_subcores, info.num_lanes
    nw = nc * ns
    bpw = B // nw

    mesh = plsc.VectorSubcoreMesh(core_axis_name="c", subcore_axis_name="s")

    @functools.partial(
        pl.kernel,
        mesh=mesh,
        out_type=jax.ShapeDtypeStruct((NPAIR * B,), jnp.float32),
        scratch_types=[
            pltpu.VMEM((bpw,), jnp.int32),
            pltpu.VMEM((NPAIR * bpw,), jnp.int32),
            pltpu.VMEM((NPAIR * bpw,), jnp.float32),
            pltpu.SemaphoreType.DMA,
        ],
    )
    def gather_k(tablef_hbm, idx_hbm, out_hbm, idx_v, idxall_v, dst_v, sem):
        wid = lax.axis_index("s") * nc + lax.axis_index("c")
        base = wid * bpw
        pltpu.sync_copy(idx_hbm.at[pl.ds(base, bpw)], idx_v)

        def body(k, carry):
            sl = idx_v[pl.ds(k * nl, nl)]
            for p in range(NPAIR):
                idxall_v[pl.ds(p * bpw + k * nl, nl)] = sl + p * SLAB
            return carry

        lax.fori_loop(0, bpw // nl, body, 0)
        pltpu.async_copy(tablef_hbm.at[idxall_v], dst_v, sem).wait()
        for p in range(NPAIR):
            pltpu.sync_copy(dst_v.at[pl.ds(p * bpw, bpw)],
                            out_hbm.at[pl.ds(p * B + base, bpw)])

    return gather_k


def _mlp_body(xt_ref, embp_ref, w1_ref, b1_ref, w2t_ref, b2_ref, out_ref):
    dn = (((0,), (0,)), ((), ()))
    x = xt_ref[:IDX, :]
    u = pltpu.bitcast(embp_ref[...], jnp.uint32)
    ev = pltpu.bitcast(u << 16, jnp.float32)
    od = pltpu.bitcast(u & jnp.uint32(0xFFFF0000), jnp.float32)
    h = lax.dot_general(w1_ref[:IDX, :], x, dn,
                        preferred_element_type=jnp.float32)
    h = h + lax.dot_general(w1_ref[IDX:IDX + NPAIR, :], ev, dn,
                            preferred_element_type=jnp.float32)
    h = h + lax.dot_general(w1_ref[IDX + NPAIR:, :], od, dn,
                            preferred_element_type=jnp.float32)
    h = jnp.maximum(h + b1_ref[...], 0.0)
    o = lax.dot_general(w2t_ref[...], h, (((1,), (0,)), ((), ())),
                        preferred_element_type=jnp.float32)
    out_ref[...] = jnp.maximum(o + b2_ref[...], 0.0)


def _mlp(xt, embp, w1, b1c, w2t, b2c):
    grid = (B // BLK,)
    return pl.pallas_call(
        _mlp_body,
        grid=grid,
        in_specs=[
            pl.BlockSpec((F, BLK), lambda i: (0, i)),
            pl.BlockSpec((NPAIR, BLK), lambda i: (0, i)),
            pl.BlockSpec((IDX + EMB, HID), lambda i: (0, 0)),
            pl.BlockSpec((HID, 1), lambda i: (0, 0)),
            pl.BlockSpec((OUT, HID), lambda i: (0, 0)),
            pl.BlockSpec((OUT, 1), lambda i: (0, 0)),
        ],
        out_specs=pl.BlockSpec((OUT, BLK), lambda i: (0, i)),
        out_shape=jax.ShapeDtypeStruct((OUT, B), jnp.float32),
    )(xt, embp, w1, b1c, w2t, b2c)


def kernel(inputs, table, W1, b1, W2, b2):
    tablet = table.T
    xt = inputs.T
    tablef, idx = _relayout(tablet, xt)
    embp = _build_gather()(tablef, idx).reshape(NPAIR, B)
    x_out_t = _mlp(xt, embp, W1, b1.reshape(HID, 1), W2.T,
                   b2.reshape(OUT, 1))
    return x_out_t.T.astype(jnp.complex64)[..., None]


# speedup vs baseline: 1.0147x; 1.0147x over previous
import functools

import jax
import jax.numpy as jnp
from jax import lax
from jax.experimental import pallas as pl
from jax.experimental.pallas import tpu as pltpu
from jax.experimental.pallas import tpu_sc as plsc

B = 16384
F = 27
IDX = 26
VOCAB = 1000000
EMB = 16
NPAIR = EMB // 2
HID = 128
OUT = 64

BLK = 8192
CH = 131072
SLAB = 8 * CH
NCOL = (VOCAB + CH - 1) // CH


def _rne16(v):
    return (v + jnp.uint32(0x7FFF) + ((v >> 16) & jnp.uint32(1))) & jnp.uint32(
        0xFFFF0000)


def _relayout_body(t_ref, xt_ref, o_ref, idx_ref, pk_ref, sem):
    c = pl.program_id(0)
    cur = c % 2
    @pl.when(c == 0)
    def _():
        idx_ref[...] = xt_ref[IDX, :].astype(jnp.int32)
    u = pltpu.bitcast(t_ref[...], jnp.uint32)
    lo = u[:NPAIR, :]
    hi = u[NPAIR:, :]
    packed = pltpu.bitcast(_rne16(hi) | (_rne16(lo) >> 16), jnp.float32)
    @pl.when(c >= 2)
    def _():
        for t in range(NPAIR):
            pltpu.make_async_copy(
                pk_ref.at[cur, t], o_ref.at[pl.ds(t * CH, CH)], sem).wait()
    pk_ref[pl.ds(cur, 1)] = packed.reshape(1, NPAIR, CH)
    for t in range(NPAIR):
        pltpu.make_async_copy(
            pk_ref.at[cur, t],
            o_ref.at[pl.ds(t * SLAB + c * CH, CH)], sem).start()
    @pl.when(c == NCOL - 1)
    def _():
        for t in range(2 * NPAIR):
            pltpu.make_async_copy(
                pk_ref.at[0, t % NPAIR], o_ref.at[pl.ds((t % NPAIR) * CH, CH)],
                sem).wait()


def _relayout(tablet, xt):
    return pl.pallas_call(
        _relayout_body,
        grid=(NCOL,),
        in_specs=[
            pl.BlockSpec((EMB, CH), lambda c: (0, c)),
            pl.BlockSpec((F, B), lambda c: (0, 0)),
        ],
        out_specs=[
            pl.BlockSpec(memory_space=pl.ANY),
            pl.BlockSpec((B,), lambda c: (0,)),
        ],
        out_shape=[
            jax.ShapeDtypeStruct((NPAIR * SLAB,), jnp.float32),
            jax.ShapeDtypeStruct((B,), jnp.int32),
        ],
        scratch_shapes=[
            pltpu.VMEM((2, NPAIR, CH), jnp.float32),
            pltpu.SemaphoreType.DMA,
        ],
    )(tablet, xt)


def _build_gather():
    info = plsc.get_sparse_core_info()
    nc, ns, nl = info.num_cores, info.num_subcores, info.num_lanes
    nw = nc * ns
    bpw = B // nw

    mesh = plsc.VectorSubcoreMesh(core_axis_name="c", subcore_axis_name="s")

    @functools.partial(
        pl.kernel,
        mesh=mesh,
        out_type=jax.ShapeDtypeStruct((NPAIR * B,), jnp.float32),
        scratch_types=[
            pltpu.VMEM((bpw,), jnp.int32),
            pltpu.VMEM((NPAIR * bpw,), jnp.int32),
            pltpu.VMEM((NPAIR * bpw,), jnp.float32),
            pltpu.SemaphoreType.DMA,
        ],
    )
    def gather_k(tablef_hbm, idx_hbm, out_hbm, idx_v, idxall_v, dst_v, sem):
        wid = lax.axis_index("s") * nc + lax.axis_index("c")
        base = wid * bpw
        pltpu.sync_copy(idx_hbm.at[pl.ds(base, bpw)], idx_v)

        def body(k, carry):
            sl = idx_v[pl.ds(k * nl, nl)]
            for p in range(NPAIR):
                idxall_v[pl.ds(p * bpw + k * nl, nl)] = sl + p * SLAB
            return carry

        lax.fori_loop(0, bpw // nl, body, 0)
        pltpu.async_copy(tablef_hbm.at[idxall_v], dst_v, sem).wait()
        for p in range(NPAIR):
            pltpu.sync_copy(dst_v.at[pl.ds(p * bpw, bpw)],
                            out_hbm.at[pl.ds(p * B + base, bpw)])

    return gather_k


def _mlp_body(xt_ref, embp_ref, w1_ref, b1_ref, w2t_ref, b2_ref, out_ref):
    dn = (((0,), (0,)), ((), ()))
    x = xt_ref[:IDX, :]
    u = pltpu.bitcast(embp_ref[...], jnp.uint32)
    ev = pltpu.bitcast(u << 16, jnp.float32)
    od = pltpu.bitcast(u & jnp.uint32(0xFFFF0000), jnp.float32)
    h = lax.dot_general(w1_ref[:IDX, :], x, dn,
                        preferred_element_type=jnp.float32)
    h = h + lax.dot_general(w1_ref[IDX:IDX + NPAIR, :], ev, dn,
                            preferred_element_type=jnp.float32)
    h = h + lax.dot_general(w1_ref[IDX + NPAIR:, :], od, dn,
                            preferred_element_type=jnp.float32)
    h = jnp.maximum(h + b1_ref[...], 0.0)
    o = lax.dot_general(w2t_ref[...], h, (((1,), (0,)), ((), ())),
                        preferred_element_type=jnp.float32)
    out_ref[...] = jnp.maximum(o + b2_ref[...], 0.0)


def _mlp(xt, embp, w1, b1c, w2t, b2c):
    grid = (B // BLK,)
    return pl.pallas_call(
        _mlp_body,
        grid=grid,
        in_specs=[
            pl.BlockSpec((F, BLK), lambda i: (0, i)),
            pl.BlockSpec((NPAIR, BLK), lambda i: (0, i)),
            pl.BlockSpec((IDX + EMB, HID), lambda i: (0, 0)),
            pl.BlockSpec((HID, 1), lambda i: (0, 0)),
            pl.BlockSpec((OUT, HID), lambda i: (0, 0)),
            pl.BlockSpec((OUT, 1), lambda i: (0, 0)),
        ],
        out_specs=pl.BlockSpec((OUT, BLK), lambda i: (0, i)),
        out_shape=jax.ShapeDtypeStruct((OUT, B), jnp.float32),
    )(xt, embp, w1, b1c, w2t, b2c)


def kernel(inputs, table, W1, b1, W2, b2):
    tablet = table.T
    xt = inputs.T
    tablef, idx = _relayout(tablet, xt)
    embp = _build_gather()(tablef, idx).reshape(NPAIR, B)
    x_out_t = _mlp(xt, embp, W1, b1.reshape(HID, 1), W2.T,
                   b2.reshape(OUT, 1))
    return x_out_t.T.astype(jnp.complex64)[..., None]
